# Initial kernel scaffold; baseline (speedup 1.0000x reference)
import jax
import jax.numpy as jnp
from jax.experimental import pallas as pl


def kernel(state_ids, action_ids, state_table, action_table):
    raise NotImplementedError("write your pallas kernel here")


# speedup vs baseline: 1.9623x; 1.9623x over previous
import functools

import jax
import jax.numpy as jnp
from jax import lax
from jax.experimental import pallas as pl
from jax.experimental.pallas import tpu as pltpu
from jax.experimental.pallas import tpu_sc as plsc

_NC = 2
_NS = 16
_NW = _NC * _NS
_L = 16

_B = 16384
_D = 64
_BPW = _B // _NW
_CH = 128
_NCH = _BPW // _CH


def _make_path_emb():
    mesh = plsc.VectorSubcoreMesh(core_axis_name="c", subcore_axis_name="s")

    @functools.partial(
        pl.kernel,
        out_type=jax.ShapeDtypeStruct((_B, _D), jnp.float32),
        mesh=mesh,
        compiler_params=pltpu.CompilerParams(use_tc_tiling_on_sc=False),
        scratch_types=[
            pltpu.VMEM((_NCH, _CH), jnp.int32),
            pltpu.VMEM((_NCH, _CH), jnp.int32),
            pltpu.VMEM((_BPW, _D), jnp.float32),
            pltpu.VMEM((_BPW, _D), jnp.float32),
            pltpu.SemaphoreType.DMA,
            pltpu.SemaphoreType.DMA,
        ],
    )
    def k(sid_hbm, aid_hbm, stab_hbm, atab_hbm, out_hbm,
          sidx_v, aidx_v, srow_v, arow_v, sem_s, sem_a):
        wid = lax.axis_index("s") * _NC + lax.axis_index("c")
        base = wid * _BPW
        pltpu.sync_copy(sid_hbm.at[wid], sidx_v)
        pltpu.sync_copy(aid_hbm.at[wid], aidx_v)
        copies = []
        for j in range(_NCH):
            dst = pl.ds(j * _CH, _CH)
            copies.append(
                pltpu.async_copy(stab_hbm.at[sidx_v.at[j]], srow_v.at[dst], sem_s))
            copies.append(
                pltpu.async_copy(atab_hbm.at[aidx_v.at[j]], arow_v.at[dst], sem_a))
        for c in copies:
            c.wait()

        def body(i, carry):
            for r in range(4):
                row = i * 4 + r
                for c0 in range(_D // _L):
                    sl = pl.ds(c0 * _L, _L)
                    srow_v[row, sl] = srow_v[row, sl] + arow_v[row, sl]
            return carry

        lax.fori_loop(0, _BPW // 4, body, 0)

        pltpu.sync_copy(srow_v, out_hbm.at[pl.ds(base, _BPW)])

    return k


_path_emb = _make_path_emb()


def kernel(state_ids, action_ids, state_table, action_table):
    sid = state_ids.reshape(_NW, _NCH, _CH)
    aid = action_ids.reshape(_NW, _NCH, _CH)
    return _path_emb(sid, aid, state_table, action_table)


# speedup vs baseline: 1.9953x; 1.0168x over previous
import functools

import jax
import jax.numpy as jnp
from jax import lax
from jax.experimental import pallas as pl
from jax.experimental.pallas import tpu as pltpu
from jax.experimental.pallas import tpu_sc as plsc

_NC = 2
_NS = 16
_NW = _NC * _NS
_L = 16

_B = 16384
_D = 64
_BPW = _B // _NW
_CH = 128
_NCH = _BPW // _CH


def _make_path_emb():
    mesh = plsc.VectorSubcoreMesh(core_axis_name="c", subcore_axis_name="s")

    @functools.partial(
        pl.kernel,
        out_type=jax.ShapeDtypeStruct((_B, _D), jnp.float32),
        mesh=mesh,
        compiler_params=pltpu.CompilerParams(use_tc_tiling_on_sc=False),
        scratch_types=[
            pltpu.VMEM((_NCH, _CH), jnp.int32),
            pltpu.VMEM((_NCH, _CH), jnp.int32),
            pltpu.VMEM((_BPW, _D), jnp.float32),
            pltpu.VMEM((_BPW, _D), jnp.float32),
        ] + [pltpu.SemaphoreType.DMA] * _NCH + [pltpu.SemaphoreType.DMA],
    )
    def k(sid_hbm, aid_hbm, stab_hbm, atab_hbm, out_hbm,
          sidx_v, aidx_v, srow_v, arow_v, *sems):
        chunk_sems, sem_out = sems[:_NCH], sems[_NCH]
        wid = lax.axis_index("s") * _NC + lax.axis_index("c")
        base = wid * _BPW
        pltpu.sync_copy(sid_hbm.at[wid], sidx_v)
        pltpu.sync_copy(aid_hbm.at[wid], aidx_v)
        gathers = []
        for j in range(_NCH):
            dst = pl.ds(j * _CH, _CH)
            gathers.append((
                pltpu.async_copy(stab_hbm.at[sidx_v.at[j]], srow_v.at[dst],
                                 chunk_sems[j]),
                pltpu.async_copy(atab_hbm.at[aidx_v.at[j]], arow_v.at[dst],
                                 chunk_sems[j]),
            ))

        out_copies = []
        for j in range(_NCH):
            gathers[j][0].wait()
            gathers[j][1].wait()

            def body(r, carry, _j=j):
                row = _j * _CH + r
                for c0 in range(_D // _L):
                    sl = pl.ds(c0 * _L, _L)
                    plsc.addupdate(srow_v.at[row, sl], arow_v[row, sl])
                return carry

            lax.fori_loop(0, _CH, body, 0)
            chunk = pl.ds(j * _CH, _CH)
            out_copies.append(
                pltpu.async_copy(srow_v.at[chunk],
                                 out_hbm.at[pl.ds(base + j * _CH, _CH)],
                                 sem_out))
        for c in out_copies:
            c.wait()

    return k


_path_emb = _make_path_emb()


def kernel(state_ids, action_ids, state_table, action_table):
    sid = state_ids.reshape(_NW, _NCH, _CH)
    aid = action_ids.reshape(_NW, _NCH, _CH)
    return _path_emb(sid, aid, state_table, action_table)
